# baseline (speedup 1.0000x reference)
import functools

import jax
import jax.numpy as jnp
from jax import lax
from jax.experimental import pallas as pl
from jax.experimental.pallas import tpu as pltpu
from jax.experimental.pallas import tpu_sc as plsc

VOCAB = 1000000
SEQ_LEN = 200
EMBED_DIM = 64
BATCH = 4096

NUM_CORES = 2
NUM_SUBCORES = 16
NUM_WORKERS = NUM_CORES * NUM_SUBCORES
B_PER_W = BATCH // NUM_WORKERS
CHUNK = 40
G_PER_B = SEQ_LEN // CHUNK
ROWS_PER_W = B_PER_W * SEQ_LEN
SCALE = 8.0


def _make_kernel():
    mesh = plsc.VectorSubcoreMesh(core_axis_name="c", subcore_axis_name="s")

    @functools.partial(
        pl.kernel,
        mesh=mesh,
        out_type=jax.ShapeDtypeStruct((BATCH, SEQ_LEN, EMBED_DIM), jnp.float32),
        scratch_types=[
            pltpu.VMEM((B_PER_W * G_PER_B, CHUNK), jnp.int32),
            pltpu.VMEM((SEQ_LEN, EMBED_DIM), jnp.float32),
            pltpu.VMEM((SEQ_LEN, EMBED_DIM), jnp.float32),
            pltpu.SemaphoreType.DMA,
        ],
        compiler_params=pltpu.CompilerParams(use_tc_tiling_on_sc=False),
    )
    def k(idx_hbm, tok_hbm, pos_hbm, out_hbm, idx_v, pos_v, rows_v, sem):
        wid = lax.axis_index("s") * NUM_CORES + lax.axis_index("c")

        pltpu.sync_copy(idx_hbm.at[wid], idx_v)
        pltpu.sync_copy(pos_hbm, pos_v)

        def per_batch(b, _):
            copies = []
            for g in range(G_PER_B):
                copies.append(pltpu.async_copy(
                    tok_hbm.at[idx_v.at[b * G_PER_B + g]],
                    rows_v.at[pl.ds(g * CHUNK, CHUNK)],
                    sem,
                ))
            for c in copies:
                c.wait()

            def per_row(r, _):
                for g in range(EMBED_DIM // 16):
                    sl = pl.ds(g * 16, 16)
                    rows_v[r, sl] = rows_v[r, sl] * SCALE + pos_v[r, sl]
                return 0
            lax.fori_loop(0, SEQ_LEN, per_row, 0)

            pltpu.sync_copy(rows_v, out_hbm.at[wid * B_PER_W + b])
            return 0

        lax.fori_loop(0, B_PER_W, per_batch, 0)

    return k


_kernel = _make_kernel()


@jax.jit
def kernel(inputs, tok_table, pos_table):
    idx3 = inputs.reshape(NUM_WORKERS, B_PER_W * G_PER_B, CHUNK)
    return _kernel(idx3, tok_table, pos_table)
